# Initial kernel scaffold; baseline (speedup 1.0000x reference)
import jax
import jax.numpy as jnp
from jax.experimental import pallas as pl


def kernel(titles, tokens, title_table, text_table):
    raise NotImplementedError("write your pallas kernel here")


# speedup vs baseline: 6.3820x; 6.3820x over previous
import functools

import jax
import jax.numpy as jnp
from jax import lax
from jax.experimental import pallas as pl
from jax.experimental.pallas import tpu as pltpu
from jax.experimental.pallas import tpu_sc as plsc

B = 16384
L = 20
D = 128
D_OUT = 2 * D
MAX_TOKENS = 10000
ZERO_ROW = MAX_TOKENS

NUM_WORKERS = 32
ROWS_PER_W = B // NUM_WORKERS
CHUNK = 16
N_CHUNKS = ROWS_PER_W // CHUNK
LANES = 16


def _body(titles_hbm, tokens_hbm, title_tab, text_tab, out_hbm,
          tokbuf, tidx, tokrows, trows, outbuf, sem_tok, sem_ttl):
    wid = lax.axis_index("s") * 2 + lax.axis_index("c")
    base = wid * ROWS_PER_W

    def chunk_body(c, _):
        row0 = base + c * CHUNK

        pltpu.sync_copy(tokens_hbm.at[pl.ds(row0 * L, CHUNK * L)], tokbuf)
        for k in range(CHUNK * L // LANES):
            tv = tokbuf[pl.ds(k * LANES, LANES)]
            tokbuf[pl.ds(k * LANES, LANES)] = jnp.where(tv == 0, ZERO_ROW, tv)

        cp_tok = pltpu.async_copy(text_tab.at[tokbuf], tokrows, sem_tok)
        pltpu.sync_copy(titles_hbm.at[pl.ds(row0, CHUNK)], tidx)
        cp_ttl = pltpu.async_copy(title_tab.at[tidx], trows, sem_ttl)

        cp_tok.wait()
        cp_ttl.wait()

        lane = lax.iota(jnp.int32, LANES)

        def row_body(r, _):
            a = tokbuf[pl.ds(r * L, LANES)]
            b = tokbuf[pl.ds(r * L + (L - LANES), LANES)]
            ca = jnp.sum(jnp.where(a != ZERO_ROW, 1.0, 0.0))
            cb = jnp.sum(jnp.where((lane >= 2 * LANES - L) & (b != ZERO_ROW),
                                   1.0, 0.0))
            cntv = jnp.full((LANES,), ca + cb, jnp.float32)
            rec = 1.0 / jnp.maximum(cntv, 1.0)
            for j in range(D // LANES):
                s = tokrows[r * L, pl.ds(j * LANES, LANES)]
                for t in range(1, L):
                    s = s + tokrows[r * L + t, pl.ds(j * LANES, LANES)]
                outbuf[r, pl.ds(j * LANES, LANES)] = trows[r, pl.ds(j * LANES, LANES)]
                outbuf[r, pl.ds(D + j * LANES, LANES)] = s * rec
            return 0

        lax.fori_loop(0, CHUNK, row_body, 0)
        pltpu.sync_copy(outbuf, out_hbm.at[pl.ds(row0, CHUNK)])
        return 0

    lax.fori_loop(0, N_CHUNKS, chunk_body, 0)


@functools.partial(jax.jit, static_argnums=())
def _sc_call(titles_i, tokens_i, title_table, text_pad):
    mesh = plsc.VectorSubcoreMesh(core_axis_name="c", subcore_axis_name="s")
    return pl.kernel(
        _body,
        out_type=jax.ShapeDtypeStruct((B, D_OUT), jnp.float32),
        mesh=mesh,
        scratch_types=[
            pltpu.VMEM((CHUNK * L,), jnp.int32),
            pltpu.VMEM((CHUNK,), jnp.int32),
            pltpu.VMEM((CHUNK * L, D), jnp.float32),
            pltpu.VMEM((CHUNK, D), jnp.float32),
            pltpu.VMEM((CHUNK, D_OUT), jnp.float32),
            pltpu.SemaphoreType.DMA,
            pltpu.SemaphoreType.DMA,
        ],
        compiler_params=pltpu.CompilerParams(needs_layout_passes=False),
    )(titles_i, tokens_i, title_table, text_pad)


def kernel(titles, tokens, title_table, text_table):
    titles_i = titles.astype(jnp.int32)
    tokens_i = tokens.reshape(-1).astype(jnp.int32)
    text_pad = jnp.concatenate(
        [text_table, jnp.zeros((1, D), text_table.dtype)], axis=0)
    return _sc_call(titles_i, tokens_i, title_table, text_pad)


# speedup vs baseline: 8.4327x; 1.3213x over previous
import functools

import jax
import jax.numpy as jnp
from jax import lax
from jax.experimental import pallas as pl
from jax.experimental.pallas import tpu as pltpu
from jax.experimental.pallas import tpu_sc as plsc

B = 16384
L = 20
D = 128
D_OUT = 2 * D
MAX_TOKENS = 10000
ZERO_ROW = MAX_TOKENS

NUM_WORKERS = 32
ROWS_PER_W = B // NUM_WORKERS
CHUNK = 16
N_CHUNKS = ROWS_PER_W // CHUNK
LANES = 16
CL = CHUNK * L


def _body(titles_hbm, tokens_hbm, title_tab, text_tab, out_hbm,
          tokbuf0, tokbuf1, tidx0, tidx1, tokrows0, tokrows1,
          trows0, trows1, outbuf0, outbuf1,
          sem_tok0, sem_tok1, sem_ttl0, sem_ttl1):
    tokbuf = (tokbuf0, tokbuf1)
    tidx = (tidx0, tidx1)
    tokrows = (tokrows0, tokrows1)
    trows = (trows0, trows1)
    outbuf = (outbuf0, outbuf1)
    sem_tok = (sem_tok0, sem_tok1)
    sem_ttl = (sem_ttl0, sem_ttl1)

    wid = lax.axis_index("s") * 2 + lax.axis_index("c")
    base = wid * ROWS_PER_W
    lane = lax.iota(jnp.int32, LANES)

    def prefetch(chunk, p):
        row0 = base + chunk * CHUNK
        pltpu.sync_copy(tokens_hbm.at[pl.ds(row0 * L, CL)], tokbuf[p])
        for k in range(CL // LANES):
            tv = tokbuf[p][pl.ds(k * LANES, LANES)]
            tokbuf[p][pl.ds(k * LANES, LANES)] = \
                jnp.where(tv == 0, ZERO_ROW, tv)
        pltpu.async_copy(text_tab.at[tokbuf[p]], tokrows[p], sem_tok[p])
        pltpu.sync_copy(titles_hbm.at[pl.ds(row0, CHUNK)], tidx[p])
        pltpu.async_copy(title_tab.at[tidx[p]], trows[p], sem_ttl[p])

    def compute(chunk, p):
        row0 = base + chunk * CHUNK
        pltpu.make_async_copy(text_tab.at[tokbuf[p]], tokrows[p],
                              sem_tok[p]).wait()
        pltpu.make_async_copy(title_tab.at[tidx[p]], trows[p],
                              sem_ttl[p]).wait()

        def row_body(r, _):
            a = tokbuf[p][pl.ds(r * L, LANES)]
            b = tokbuf[p][pl.ds(r * L + (L - LANES), LANES)]
            ca = jnp.sum(jnp.where(a != ZERO_ROW, 1.0, 0.0))
            cb = jnp.sum(jnp.where((lane >= 2 * LANES - L) & (b != ZERO_ROW),
                                   1.0, 0.0))
            cntv = jnp.full((LANES,), ca + cb, jnp.float32)
            rec = 1.0 / jnp.maximum(cntv, 1.0)
            for j in range(D // LANES):
                s = tokrows[p][r * L, pl.ds(j * LANES, LANES)]
                for t in range(1, L):
                    s = s + tokrows[p][r * L + t, pl.ds(j * LANES, LANES)]
                outbuf[p][r, pl.ds(j * LANES, LANES)] = \
                    trows[p][r, pl.ds(j * LANES, LANES)]
                outbuf[p][r, pl.ds(D + j * LANES, LANES)] = s * rec
            return 0

        lax.fori_loop(0, CHUNK, row_body, 0)
        pltpu.sync_copy(outbuf[p], out_hbm.at[pl.ds(row0, CHUNK)])

    prefetch(0, 0)

    def outer(i, _):
        for p in range(2):
            chunk = i * 2 + p

            @pl.when(chunk + 1 < N_CHUNKS)
            def _():
                prefetch(chunk + 1, 1 - p)

            compute(chunk, p)
        return 0

    lax.fori_loop(0, N_CHUNKS // 2, outer, 0)


@functools.partial(jax.jit, static_argnums=())
def _sc_call(titles_i, tokens_i, title_table, text_pad):
    mesh = plsc.VectorSubcoreMesh(core_axis_name="c", subcore_axis_name="s")
    return pl.kernel(
        _body,
        out_type=jax.ShapeDtypeStruct((B, D_OUT), jnp.float32),
        mesh=mesh,
        scratch_types=[
            pltpu.VMEM((CL,), jnp.int32),
            pltpu.VMEM((CL,), jnp.int32),
            pltpu.VMEM((CHUNK,), jnp.int32),
            pltpu.VMEM((CHUNK,), jnp.int32),
            pltpu.VMEM((CL, D), jnp.float32),
            pltpu.VMEM((CL, D), jnp.float32),
            pltpu.VMEM((CHUNK, D), jnp.float32),
            pltpu.VMEM((CHUNK, D), jnp.float32),
            pltpu.VMEM((CHUNK, D_OUT), jnp.float32),
            pltpu.VMEM((CHUNK, D_OUT), jnp.float32),
            pltpu.SemaphoreType.DMA,
            pltpu.SemaphoreType.DMA,
            pltpu.SemaphoreType.DMA,
            pltpu.SemaphoreType.DMA,
        ],
        compiler_params=pltpu.CompilerParams(needs_layout_passes=False),
    )(titles_i, tokens_i, title_table, text_pad)


def kernel(titles, tokens, title_table, text_table):
    titles_i = titles.astype(jnp.int32)
    tokens_i = tokens.reshape(-1).astype(jnp.int32)
    text_pad = jnp.concatenate(
        [text_table, jnp.zeros((1, D), text_table.dtype)], axis=0)
    return _sc_call(titles_i, tokens_i, title_table, text_pad)
